# initial kernel scaffold (unmeasured)
import jax
import jax.numpy as jnp
from jax import lax
from jax.experimental import pallas as pl
from jax.experimental.pallas import tpu as pltpu

N_DEV = 8
N_LAYERS = 3


def kernel(x, Win0, Wout0, Win1, Wout1, Win2, Wout2):
    b, d = x.shape

    def body(
        x_ref,
        win0,
        wout0,
        win1,
        wout1,
        win2,
        wout2,
        out_ref,
        send_buf,
        comm_ref,
        send_sems,
        recv_sems,
    ):
        my = lax.axis_index("i")
        wins = [win0, win1, win2]
        wouts = [wout0, wout1, wout2]

        cur = x_ref[:, :]
        for layer in range(N_LAYERS):
            h = jnp.maximum(
                jnp.dot(cur, wins[layer][:, :], preferred_element_type=jnp.float32),
                0.0,
            )
            partial = jnp.dot(
                h, wouts[layer][:, :], preferred_element_type=jnp.float32
            )
            send_buf[layer, :, :] = partial

            sends = []
            for k in range(1, N_DEV):
                rdma = pltpu.make_async_remote_copy(
                    src_ref=send_buf.at[layer],
                    dst_ref=comm_ref.at[layer, k - 1],
                    send_sem=send_sems.at[layer, k - 1],
                    recv_sem=recv_sems.at[layer, k - 1],
                    device_id=((my + k) % N_DEV,),
                    device_id_type=pl.DeviceIdType.MESH,
                )
                rdma.start()
                sends.append(rdma)

            acc = partial
            for k in range(1, N_DEV):
                sends[k - 1].wait_recv()
                acc = acc + comm_ref[layer, k - 1, :, :]
            for k in range(1, N_DEV):
                sends[k - 1].wait_send()
            cur = acc

        out_ref[:, :] = cur

    return pl.pallas_call(
        body,
        out_shape=jax.ShapeDtypeStruct((b, d), jnp.float32),
        in_specs=[pl.BlockSpec(memory_space=pltpu.VMEM)] * 7,
        out_specs=pl.BlockSpec(memory_space=pltpu.VMEM),
        scratch_shapes=[
            pltpu.VMEM((N_LAYERS, b, d), jnp.float32),
            pltpu.VMEM((N_LAYERS, N_DEV - 1, b, d), jnp.float32),
            pltpu.SemaphoreType.DMA((N_LAYERS, N_DEV - 1)),
            pltpu.SemaphoreType.DMA((N_LAYERS, N_DEV - 1)),
        ],
        compiler_params=pltpu.CompilerParams(collective_id=0),
    )(x, Win0, Wout0, Win1, Wout1, Win2, Wout2)


# baseline (device time: 45229 ns/iter reference)
import jax
import jax.numpy as jnp
from jax import lax
from jax.experimental import pallas as pl
from jax.experimental.pallas import tpu as pltpu

N_DEV = 8
N_LAYERS = 3


def kernel(x, Win0, Wout0, Win1, Wout1, Win2, Wout2):
    b, d = x.shape

    def body(
        x_ref,
        win0,
        wout0,
        win1,
        wout1,
        win2,
        wout2,
        out_ref,
        send_buf,
        comm_ref,
        send_sems,
        recv_sems,
    ):
        my = lax.axis_index("i")
        wins = [win0, win1, win2]
        wouts = [wout0, wout1, wout2]

        cur = x_ref[:, :]
        for layer in range(N_LAYERS):
            h = jnp.maximum(
                jnp.dot(cur, wins[layer][:, :], preferred_element_type=jnp.float32),
                0.0,
            )
            partial = jnp.dot(
                h, wouts[layer][:, :], preferred_element_type=jnp.float32
            )
            send_buf[layer, :, :] = partial

            sends = []
            for k in range(1, N_DEV):
                rdma = pltpu.make_async_remote_copy(
                    src_ref=send_buf.at[layer],
                    dst_ref=comm_ref.at[layer, k - 1],
                    send_sem=send_sems.at[layer, k - 1],
                    recv_sem=recv_sems.at[layer, k - 1],
                    device_id=((my + k) % N_DEV,),
                    device_id_type=pl.DeviceIdType.MESH,
                )
                rdma.start()
                sends.append(rdma)

            acc = partial
            for k in range(1, N_DEV):
                sends[k - 1].wait_recv()
                acc = acc + comm_ref[layer, k - 1, :, :]
            for k in range(1, N_DEV):
                sends[k - 1].wait_send()
            cur = acc

        out_ref[:, :] = cur

    return pl.pallas_call(
        body,
        out_shape=jax.ShapeDtypeStruct((b, d), jnp.float32),
        in_specs=[pl.BlockSpec(memory_space=pltpu.VMEM)] * 7,
        out_specs=pl.BlockSpec(memory_space=pltpu.VMEM),
        scratch_shapes=[
            pltpu.VMEM((N_LAYERS, b, d), jnp.float32),
            pltpu.VMEM((N_LAYERS, N_DEV - 1, b, d), jnp.float32),
            pltpu.SemaphoreType.DMA((N_LAYERS, N_DEV - 1)),
            pltpu.SemaphoreType.DMA((N_LAYERS, N_DEV - 1)),
        ],
    )(x, Win0, Wout0, Win1, Wout1, Win2, Wout2)


# device time: 38503 ns/iter; 1.1747x vs baseline; 1.1747x over previous
import jax
import jax.numpy as jnp
from jax import lax
from jax.experimental import pallas as pl
from jax.experimental.pallas import tpu as pltpu

N_DEV = 8
N_LAYERS = 3


def kernel(x, Win0, Wout0, Win1, Wout1, Win2, Wout2):
    b, d = x.shape
    ch = b // N_DEV
    hsh = Win0.shape[1]

    def body(
        x_ref,
        win0_hbm,
        wout0_hbm,
        win1_hbm,
        wout1_hbm,
        win2_hbm,
        wout2_hbm,
        out_ref,
        win_v,
        wout_v,
        send_buf,
        rs_comm,
        x_buf,
        w_sems,
        rs_send_sems,
        rs_recv_sems,
        ag_send_sems,
        ag_recv_sems,
    ):
        my = lax.axis_index("i")
        wins_hbm = [win0_hbm, win1_hbm, win2_hbm]
        wouts_hbm = [wout0_hbm, wout1_hbm, wout2_hbm]

        w_copies = []
        for layer in range(N_LAYERS):
            cin = pltpu.make_async_copy(
                wins_hbm[layer], win_v.at[layer], w_sems.at[layer, 0]
            )
            cout = pltpu.make_async_copy(
                wouts_hbm[layer], wout_v.at[layer], w_sems.at[layer, 1]
            )
            cin.start()
            cout.start()
            w_copies.append((cin, cout))

        barrier_sem = pltpu.get_barrier_semaphore()
        for k in range(1, N_DEV):
            pl.semaphore_signal(
                barrier_sem,
                inc=1,
                device_id=((my + k) % N_DEV,),
                device_id_type=pl.DeviceIdType.MESH,
            )
        pl.semaphore_wait(barrier_sem, N_DEV - 1)

        cur = x_ref[:, :]
        for layer in range(N_LAYERS):
            w_copies[layer][0].wait()
            h = jnp.maximum(
                jnp.dot(
                    cur, win_v[layer, :, :], preferred_element_type=jnp.float32
                ),
                0.0,
            )
            w_copies[layer][1].wait()
            partial = jnp.dot(
                h, wout_v[layer, :, :], preferred_element_type=jnp.float32
            )
            send_buf[layer, :, :] = partial

            rs = []
            for k in range(1, N_DEV):
                p = (my + k) % N_DEV
                rdma = pltpu.make_async_remote_copy(
                    src_ref=send_buf.at[layer, pl.ds(p * ch, ch), :],
                    dst_ref=rs_comm.at[layer, k - 1],
                    send_sem=rs_send_sems.at[layer, k - 1],
                    recv_sem=rs_recv_sems.at[layer, k - 1],
                    device_id=(p,),
                    device_id_type=pl.DeviceIdType.MESH,
                )
                rdma.start()
                rs.append(rdma)

            red = send_buf[layer, pl.ds(my * ch, ch), :]
            for k in range(1, N_DEV):
                rs[k - 1].wait_recv()
                red = red + rs_comm[layer, k - 1, :, :]
            x_buf[layer, pl.ds(my * ch, ch), :] = red

            ag = []
            for k in range(1, N_DEV):
                p = (my + k) % N_DEV
                rdma = pltpu.make_async_remote_copy(
                    src_ref=x_buf.at[layer, pl.ds(my * ch, ch), :],
                    dst_ref=x_buf.at[layer, pl.ds(my * ch, ch), :],
                    send_sem=ag_send_sems.at[layer, k - 1],
                    recv_sem=ag_recv_sems.at[layer, k - 1],
                    device_id=(p,),
                    device_id_type=pl.DeviceIdType.MESH,
                )
                rdma.start()
                ag.append(rdma)

            for k in range(1, N_DEV):
                ag[k - 1].wait_recv()
            for k in range(1, N_DEV):
                rs[k - 1].wait_send()
                ag[k - 1].wait_send()
            cur = x_buf[layer, :, :]

        out_ref[:, :] = cur

    return pl.pallas_call(
        body,
        out_shape=jax.ShapeDtypeStruct((b, d), jnp.float32),
        in_specs=[pl.BlockSpec(memory_space=pltpu.VMEM)]
        + [pl.BlockSpec(memory_space=pl.ANY)] * 6,
        out_specs=pl.BlockSpec(memory_space=pltpu.VMEM),
        scratch_shapes=[
            pltpu.VMEM((N_LAYERS, d, hsh), jnp.float32),
            pltpu.VMEM((N_LAYERS, hsh, d), jnp.float32),
            pltpu.VMEM((N_LAYERS, b, d), jnp.float32),
            pltpu.VMEM((N_LAYERS, N_DEV - 1, ch, d), jnp.float32),
            pltpu.VMEM((N_LAYERS, b, d), jnp.float32),
            pltpu.SemaphoreType.DMA((N_LAYERS, 2)),
            pltpu.SemaphoreType.DMA((N_LAYERS, N_DEV - 1)),
            pltpu.SemaphoreType.DMA((N_LAYERS, N_DEV - 1)),
            pltpu.SemaphoreType.DMA((N_LAYERS, N_DEV - 1)),
            pltpu.SemaphoreType.DMA((N_LAYERS, N_DEV - 1)),
        ],
        compiler_params=pltpu.CompilerParams(collective_id=0),
    )(x, Win0, Wout0, Win1, Wout1, Win2, Wout2)
